# Initial kernel scaffold; baseline (speedup 1.0000x reference)
import jax
import jax.numpy as jnp
from jax.experimental import pallas as pl


def kernel(x, table):
    raise NotImplementedError("write your pallas kernel here")


# speedup vs baseline: 1.2498x; 1.2498x over previous
import jax
import jax.numpy as jnp
from jax.experimental import pallas as pl

D_MODEL = 64
N_BITS_PAD = 32
BLOCK = 4096


def _body(x_ref, t_ref, o_ref):
    xcol = x_ref[0]
    iot = jax.lax.broadcasted_iota(jnp.int32, (1, N_BITS_PAD), 1)
    bits = ((xcol >> iot) & 1).astype(jnp.float32)
    o_ref[0] = jnp.dot(bits, t_ref[...], preferred_element_type=jnp.float32)


def kernel(x, table):
    x_shape = x.shape
    n = x.size
    assert n % BLOCK == 0, n
    nb = n // BLOCK
    xf = x.reshape(nb, BLOCK, 1)
    tpad = jnp.zeros((N_BITS_PAD, D_MODEL), table.dtype).at[: table.shape[0]].set(table)
    out = pl.pallas_call(
        _body,
        grid=(nb,),
        in_specs=[
            pl.BlockSpec((1, BLOCK, 1), lambda i: (i, 0, 0)),
            pl.BlockSpec((N_BITS_PAD, D_MODEL), lambda i: (0, 0)),
        ],
        out_specs=pl.BlockSpec((1, BLOCK, D_MODEL), lambda i: (i, 0, 0)),
        out_shape=jax.ShapeDtypeStruct((nb, BLOCK, D_MODEL), jnp.float32),
    )(xf, tpad)
    return out.reshape(*x_shape, D_MODEL)
